# Initial kernel scaffold; baseline (speedup 1.0000x reference)
import jax
import jax.numpy as jnp
from jax.experimental import pallas as pl


def kernel(x, knn_edge_index, W1l, b1, W1r, W2l, b2, W2r):
    raise NotImplementedError("write your pallas kernel here")


# speedup vs baseline: 3.9331x; 3.9331x over previous
import dataclasses
import functools

import jax
import jax.numpy as jnp
from jax import lax
from jax.experimental import pallas as pl
from jax.experimental.pallas import tpu as pltpu
from jax.experimental.pallas import tpu_sc as plsc

N = 10000
D = 128
H = 128
E = 320000

NC = 2
NS = 16
NW = NC * NS

FW = 64
CW = 128
CPT = 160
EPAD = NS * CPT * CW
KB = 40
NB = CPT // KB
NPAD = 10240
RPT = NPAD // (NS * CW)

BLK = 512


def _sc_body(tlo, thi, srcr, dstr, z64, oagg, sidx, didx, rows, zbuf,
             acc, ocnt=None, cnt_local=None):
    cid = lax.axis_index("c")
    sid = lax.axis_index("s")
    pltpu.sync_copy(z64, zbuf)
    for r in range(RPT):
        row0 = (sid * RPT + r) * CW
        pltpu.sync_copy(zbuf, acc.at[pl.ds(row0, CW)])
    if cnt_local is not None:
        @pl.loop(0, NPAD // 16)
        def _(i):
            cnt_local[pl.ds(i * 16, 16)] = jnp.zeros((16,), jnp.float32)
    plsc.subcore_barrier()

    for b in range(NB):
        base = sid * CPT + b * KB
        pltpu.sync_copy(srcr.at[pl.ds(base, KB)], sidx)
        pltpu.sync_copy(dstr.at[pl.ds(base, KB)], didx)

        @pl.loop(0, KB)
        def _(j):
            @pl.when(cid == 0)
            def _():
                pltpu.sync_copy(tlo.at[sidx.at[j]], rows)

            @pl.when(cid == 1)
            def _():
                pltpu.sync_copy(thi.at[sidx.at[j]], rows)

            pltpu.sync_copy(rows, acc.at[didx.at[j]], add=True)

    if cnt_local is not None:
        for b in range(NB // 2):
            base = sid * CPT + (cid * (NB // 2) + b) * KB
            pltpu.sync_copy(dstr.at[pl.ds(base, KB)], didx)

            @pl.loop(0, KB)
            def _(j):
                for k16 in range(CW // 16):
                    d = didx[j, pl.ds(k16 * 16, 16)]
                    cnts, last = plsc.scan_count(d)
                    plsc.addupdate_scatter(
                        cnt_local, [d], cnts.astype(jnp.float32), mask=last)

    plsc.subcore_barrier()
    for r in range(RPT):
        row0 = (sid * RPT + r) * CW
        pltpu.sync_copy(acc.at[pl.ds(row0, CW)], zbuf)
        pltpu.sync_copy(zbuf, oagg.at[cid, pl.ds(row0, CW)])
    if cnt_local is not None:
        wid = cid * NS + sid
        pltpu.sync_copy(cnt_local, ocnt.at[pl.ds(wid * NPAD, NPAD)])


def _sc_compiler_params():
    cp = pltpu.CompilerParams(use_tc_tiling_on_sc=False)
    if "needs_layout_passes" in pltpu.CompilerParams.__dataclass_fields__:
        cp = dataclasses.replace(cp, needs_layout_passes=False)
    return cp


def _make_sc(with_counts):
    mesh = plsc.VectorSubcoreMesh(core_axis_name="c", subcore_axis_name="s")
    agg_t = jax.ShapeDtypeStruct((NC, NPAD, FW), jnp.float32)
    cnt_t = jax.ShapeDtypeStruct((NW * NPAD,), jnp.float32)
    scratch = [
        pltpu.VMEM((KB, CW), jnp.int32),
        pltpu.VMEM((KB, CW), jnp.int32),
        pltpu.VMEM((CW, FW), jnp.float32),
        pltpu.VMEM((CW, FW), jnp.float32),
        pltpu.VMEM_SHARED((NPAD, FW), jnp.float32),
    ]
    if with_counts:
        scratch.append(pltpu.VMEM((NPAD,), jnp.float32))

        @functools.partial(pl.kernel, out_type=(agg_t, cnt_t), mesh=mesh,
                           scratch_types=scratch,
                           compiler_params=_sc_compiler_params())
        def k(tlo, thi, srcr, dstr, z64, oagg, ocnt, sidx, didx, rows,
              zbuf, acc, cnt_local):
            _sc_body(tlo, thi, srcr, dstr, z64, oagg, sidx, didx, rows,
                     zbuf, acc, ocnt=ocnt, cnt_local=cnt_local)
    else:

        @functools.partial(pl.kernel, out_type=agg_t, mesh=mesh,
                           scratch_types=scratch,
                           compiler_params=_sc_compiler_params())
        def k(tlo, thi, srcr, dstr, z64, oagg, sidx, didx, rows, zbuf, acc):
            _sc_body(tlo, thi, srcr, dstr, z64, oagg, sidx, didx, rows,
                     zbuf, acc)

    return k


_sc_agg_counts = _make_sc(True)
_sc_agg_plain = _make_sc(False)


def _prep_call(xp, eye, W1r):
    def body(x_ref, e_ref, w_ref, tlo_ref, thi_ref, r_ref):
        xb = x_ref[...]
        t = lax.dot_general(xb, e_ref[...], (((0,), (0,)), ((), ())),
                            preferred_element_type=jnp.float32)
        tlo_ref[...] = t[:, :FW]
        thi_ref[...] = t[:, FW:]
        r_ref[...] = lax.dot_general(t, w_ref[...], (((1,), (1,)), ((), ())),
                                     preferred_element_type=jnp.float32)

    return pl.pallas_call(
        body,
        grid=(NPAD // BLK,),
        in_specs=[pl.BlockSpec((D, BLK), lambda i: (0, i)),
                  pl.BlockSpec((D, D), lambda i: (0, 0)),
                  pl.BlockSpec((H, D), lambda i: (0, 0))],
        out_specs=[pl.BlockSpec((BLK, FW), lambda i: (i, 0)),
                   pl.BlockSpec((BLK, FW), lambda i: (i, 0)),
                   pl.BlockSpec((BLK, H), lambda i: (i, 0))],
        out_shape=[jax.ShapeDtypeStruct((NPAD, FW), jnp.float32),
                   jax.ShapeDtypeStruct((NPAD, FW), jnp.float32),
                   jax.ShapeDtypeStruct((NPAD, H), jnp.float32)],
        compiler_params=pltpu.CompilerParams(
            dimension_semantics=("parallel",)),
    )(xp, eye, W1r)


def _elu(v):
    return jnp.where(v > 0, v, jnp.exp(jnp.minimum(v, 0.0)) - 1.0)


def _mid_call(aggp, cntp, r1, W1l, b1, W2r):
    def body(a_ref, c_ref, r_ref, wl_ref, b_ref, wn_ref,
             hlo_ref, hhi_ref, rn_ref):
        av = a_ref[...]
        cv = c_ref[...]
        a = jnp.concatenate([av[0], av[1]], axis=1)
        cnt = jnp.sum(cv, axis=1, keepdims=True)
        mean = a / jnp.maximum(cnt, 1.0)
        v = lax.dot_general(mean, wl_ref[...], (((1,), (1,)), ((), ())),
                            preferred_element_type=jnp.float32)
        h = _elu(v + b_ref[...] + r_ref[...])
        hlo_ref[...] = h[:, :FW]
        hhi_ref[...] = h[:, FW:]
        rn_ref[...] = lax.dot_general(h, wn_ref[...], (((1,), (1,)), ((), ())),
                                      preferred_element_type=jnp.float32)

    return pl.pallas_call(
        body,
        grid=(NPAD // BLK,),
        in_specs=[pl.BlockSpec((NC, BLK, FW), lambda i: (0, i, 0)),
                  pl.BlockSpec((BLK, NW), lambda i: (i, 0)),
                  pl.BlockSpec((BLK, H), lambda i: (i, 0)),
                  pl.BlockSpec((H, H), lambda i: (0, 0)),
                  pl.BlockSpec((1, H), lambda i: (0, 0)),
                  pl.BlockSpec((H, H), lambda i: (0, 0))],
        out_specs=[pl.BlockSpec((BLK, FW), lambda i: (i, 0)),
                   pl.BlockSpec((BLK, FW), lambda i: (i, 0)),
                   pl.BlockSpec((BLK, H), lambda i: (i, 0))],
        out_shape=[jax.ShapeDtypeStruct((NPAD, FW), jnp.float32),
                   jax.ShapeDtypeStruct((NPAD, FW), jnp.float32),
                   jax.ShapeDtypeStruct((NPAD, H), jnp.float32)],
        compiler_params=pltpu.CompilerParams(
            dimension_semantics=("parallel",)),
    )(aggp, cntp, r1, W1l, b1, W2r)


def _final_call(aggp, cntp, r2, W2l, b2):
    def body(a_ref, c_ref, r_ref, wl_ref, b_ref, o_ref):
        av = a_ref[...]
        cv = c_ref[...]
        a = jnp.concatenate([av[0], av[1]], axis=1)
        cnt = jnp.sum(cv, axis=1, keepdims=True)
        mean = a / jnp.maximum(cnt, 1.0)
        v = lax.dot_general(mean, wl_ref[...], (((1,), (1,)), ((), ())),
                            preferred_element_type=jnp.float32)
        o_ref[...] = _elu(v + b_ref[...] + r_ref[...])

    return pl.pallas_call(
        body,
        grid=(NPAD // BLK,),
        in_specs=[pl.BlockSpec((NC, BLK, FW), lambda i: (0, i, 0)),
                  pl.BlockSpec((BLK, NW), lambda i: (i, 0)),
                  pl.BlockSpec((BLK, H), lambda i: (i, 0)),
                  pl.BlockSpec((H, H), lambda i: (0, 0)),
                  pl.BlockSpec((1, H), lambda i: (0, 0))],
        out_specs=pl.BlockSpec((BLK, H), lambda i: (i, 0)),
        out_shape=jax.ShapeDtypeStruct((NPAD, H), jnp.float32),
        compiler_params=pltpu.CompilerParams(
            dimension_semantics=("parallel",)),
    )(aggp, cntp, r2, W2l, b2)


def kernel(x, knn_edge_index, W1l, b1, W1r, W2l, b2, W2r):
    src = knn_edge_index[0].astype(jnp.int32)
    dst = knn_edge_index[1].astype(jnp.int32)
    pad = EPAD - E
    srcp = jnp.concatenate([src, jnp.zeros((pad,), jnp.int32)]).reshape(
        NS * CPT, CW)
    dstp = jnp.concatenate([dst, jnp.full((pad,), N, jnp.int32)]).reshape(
        NS * CPT, CW)
    xp = jnp.pad(x, ((0, 0), (0, NPAD - N)))
    eye = jnp.eye(D, dtype=jnp.float32)
    z64 = jnp.zeros((CW, FW), jnp.float32)

    tlo, thi, r1 = _prep_call(xp, eye, W1r)
    agg1, cnt1 = _sc_agg_counts(tlo, thi, srcp, dstp, z64)
    cnt1t = cnt1.reshape(NW, NPAD).T
    hlo, hhi, r2 = _mid_call(agg1, cnt1t, r1, W1l, b1.reshape(1, H), W2r)
    agg2 = _sc_agg_plain(hlo, hhi, srcp, dstp, z64)
    o = _final_call(agg2, cnt1t, r2, W2l, b2.reshape(1, H))
    return o[:N]


# speedup vs baseline: 4.7787x; 1.2150x over previous
import dataclasses
import functools

import jax
import jax.numpy as jnp
from jax import lax
from jax.experimental import pallas as pl
from jax.experimental.pallas import tpu as pltpu
from jax.experimental.pallas import tpu_sc as plsc

N = 10000
D = 128
H = 128
E = 320000

NC = 2
NS = 16
NW = NC * NS

FW = 64
CW = 128
CPT = 160
EPAD = NS * CPT * CW
KB = 16
NB = CPT // KB
NBUF = 4
LAG = 2
NPAD = 10240
RPT = NPAD // (NS * CW)

BLK = 512


def _sc_body(tlo, thi, srcr, dstr, z64, oagg, sidx, didx, rows, gsems,
             ssems, acc, ocnt=None, cnt_local=None):
    cid = lax.axis_index("c")
    sid = lax.axis_index("s")
    pltpu.sync_copy(z64, rows.at[0])
    for r in range(RPT):
        row0 = (sid * RPT + r) * CW
        pltpu.sync_copy(rows.at[0], acc.at[pl.ds(row0, CW)])
    if cnt_local is not None:
        @pl.loop(0, NPAD // 16)
        def _(i):
            cnt_local[pl.ds(i * 16, 16)] = jnp.zeros((16,), jnp.float32)
    plsc.subcore_barrier()

    def gstart(j):
        @pl.when(cid == 0)
        def _():
            pltpu.async_copy(tlo.at[sidx.at[j]], rows.at[j % NBUF],
                             gsems.at[j % NBUF])

        @pl.when(cid == 1)
        def _():
            pltpu.async_copy(thi.at[sidx.at[j]], rows.at[j % NBUF],
                             gsems.at[j % NBUF])

    def gwait(j):
        pltpu.make_async_copy(tlo.at[pl.ds(0, CW)], rows.at[j % NBUF],
                              gsems.at[j % NBUF]).wait()

    def sstart(j):
        pltpu.async_copy(rows.at[j % NBUF], acc.at[didx.at[j]],
                         ssems.at[j % NBUF], add=True)

    def swait(j):
        pltpu.make_async_copy(rows.at[j % NBUF], acc.at[pl.ds(0, CW)],
                              ssems.at[j % NBUF]).wait()

    for b in range(NB):
        base = sid * CPT + b * KB
        pltpu.sync_copy(srcr.at[pl.ds(base, KB)], sidx)
        pltpu.sync_copy(dstr.at[pl.ds(base, KB)], didx)
        for j in range(NBUF):
            gstart(j)
        for j in range(KB):
            gwait(j)
            sstart(j)
            k = j - LAG
            if k >= 0 and k + NBUF < KB:
                swait(k)
                gstart(k + NBUF)
        for j in range(KB - NBUF, KB):
            swait(j)

    if cnt_local is not None:
        for b in range(NB // 2):
            base = sid * CPT + (cid * (NB // 2) + b) * KB
            pltpu.sync_copy(dstr.at[pl.ds(base, KB)], didx)

            @pl.loop(0, KB)
            def _(j):
                for k16 in range(CW // 16):
                    d = didx[j, pl.ds(k16 * 16, 16)]
                    cnts, last = plsc.scan_count(d)
                    plsc.addupdate_scatter(
                        cnt_local, [d], cnts.astype(jnp.float32), mask=last)

    plsc.subcore_barrier()
    for r in range(RPT):
        row0 = (sid * RPT + r) * CW
        pltpu.sync_copy(acc.at[pl.ds(row0, CW)], rows.at[0])
        pltpu.sync_copy(rows.at[0], oagg.at[cid, pl.ds(row0, CW)])
    if cnt_local is not None:
        wid = cid * NS + sid
        pltpu.sync_copy(cnt_local, ocnt.at[pl.ds(wid * NPAD, NPAD)])


def _sc_compiler_params():
    cp = pltpu.CompilerParams(use_tc_tiling_on_sc=False)
    if "needs_layout_passes" in pltpu.CompilerParams.__dataclass_fields__:
        cp = dataclasses.replace(cp, needs_layout_passes=False)
    return cp


def _make_sc(with_counts):
    mesh = plsc.VectorSubcoreMesh(core_axis_name="c", subcore_axis_name="s")
    agg_t = jax.ShapeDtypeStruct((NC, NPAD, FW), jnp.float32)
    cnt_t = jax.ShapeDtypeStruct((NW * NPAD,), jnp.float32)
    scratch = [
        pltpu.VMEM((KB, CW), jnp.int32),
        pltpu.VMEM((KB, CW), jnp.int32),
        pltpu.VMEM((NBUF, CW, FW), jnp.float32),
        pltpu.SemaphoreType.DMA((NBUF,)),
        pltpu.SemaphoreType.DMA((NBUF,)),
        pltpu.VMEM_SHARED((NPAD, FW), jnp.float32),
    ]
    if with_counts:
        scratch.append(pltpu.VMEM((NPAD,), jnp.float32))

        @functools.partial(pl.kernel, out_type=(agg_t, cnt_t), mesh=mesh,
                           scratch_types=scratch,
                           compiler_params=_sc_compiler_params())
        def k(tlo, thi, srcr, dstr, z64, oagg, ocnt, sidx, didx, rows,
              gsems, ssems, acc, cnt_local):
            _sc_body(tlo, thi, srcr, dstr, z64, oagg, sidx, didx, rows,
                     gsems, ssems, acc, ocnt=ocnt, cnt_local=cnt_local)
    else:

        @functools.partial(pl.kernel, out_type=agg_t, mesh=mesh,
                           scratch_types=scratch,
                           compiler_params=_sc_compiler_params())
        def k(tlo, thi, srcr, dstr, z64, oagg, sidx, didx, rows,
              gsems, ssems, acc):
            _sc_body(tlo, thi, srcr, dstr, z64, oagg, sidx, didx, rows,
                     gsems, ssems, acc)

    return k


_sc_agg_counts = _make_sc(True)
_sc_agg_plain = _make_sc(False)


def _prep_call(xp, eye, W1r):
    def body(x_ref, e_ref, w_ref, tlo_ref, thi_ref, r_ref):
        xb = x_ref[...]
        t = lax.dot_general(xb, e_ref[...], (((0,), (0,)), ((), ())),
                            preferred_element_type=jnp.float32)
        tlo_ref[...] = t[:, :FW]
        thi_ref[...] = t[:, FW:]
        r_ref[...] = lax.dot_general(t, w_ref[...], (((1,), (1,)), ((), ())),
                                     preferred_element_type=jnp.float32)

    return pl.pallas_call(
        body,
        grid=(NPAD // BLK,),
        in_specs=[pl.BlockSpec((D, BLK), lambda i: (0, i)),
                  pl.BlockSpec((D, D), lambda i: (0, 0)),
                  pl.BlockSpec((H, D), lambda i: (0, 0))],
        out_specs=[pl.BlockSpec((BLK, FW), lambda i: (i, 0)),
                   pl.BlockSpec((BLK, FW), lambda i: (i, 0)),
                   pl.BlockSpec((BLK, H), lambda i: (i, 0))],
        out_shape=[jax.ShapeDtypeStruct((NPAD, FW), jnp.float32),
                   jax.ShapeDtypeStruct((NPAD, FW), jnp.float32),
                   jax.ShapeDtypeStruct((NPAD, H), jnp.float32)],
        compiler_params=pltpu.CompilerParams(
            dimension_semantics=("parallel",)),
    )(xp, eye, W1r)


def _elu(v):
    return jnp.where(v > 0, v, jnp.exp(jnp.minimum(v, 0.0)) - 1.0)


def _mid_call(aggp, cntp, r1, W1l, b1, W2r):
    def body(a_ref, c_ref, r_ref, wl_ref, b_ref, wn_ref,
             hlo_ref, hhi_ref, rn_ref):
        av = a_ref[...]
        cv = c_ref[...]
        a = jnp.concatenate([av[0], av[1]], axis=1)
        cnt = jnp.sum(cv, axis=1, keepdims=True)
        mean = a / jnp.maximum(cnt, 1.0)
        v = lax.dot_general(mean, wl_ref[...], (((1,), (1,)), ((), ())),
                            preferred_element_type=jnp.float32)
        h = _elu(v + b_ref[...] + r_ref[...])
        hlo_ref[...] = h[:, :FW]
        hhi_ref[...] = h[:, FW:]
        rn_ref[...] = lax.dot_general(h, wn_ref[...], (((1,), (1,)), ((), ())),
                                      preferred_element_type=jnp.float32)

    return pl.pallas_call(
        body,
        grid=(NPAD // BLK,),
        in_specs=[pl.BlockSpec((NC, BLK, FW), lambda i: (0, i, 0)),
                  pl.BlockSpec((BLK, NW), lambda i: (i, 0)),
                  pl.BlockSpec((BLK, H), lambda i: (i, 0)),
                  pl.BlockSpec((H, H), lambda i: (0, 0)),
                  pl.BlockSpec((1, H), lambda i: (0, 0)),
                  pl.BlockSpec((H, H), lambda i: (0, 0))],
        out_specs=[pl.BlockSpec((BLK, FW), lambda i: (i, 0)),
                   pl.BlockSpec((BLK, FW), lambda i: (i, 0)),
                   pl.BlockSpec((BLK, H), lambda i: (i, 0))],
        out_shape=[jax.ShapeDtypeStruct((NPAD, FW), jnp.float32),
                   jax.ShapeDtypeStruct((NPAD, FW), jnp.float32),
                   jax.ShapeDtypeStruct((NPAD, H), jnp.float32)],
        compiler_params=pltpu.CompilerParams(
            dimension_semantics=("parallel",)),
    )(aggp, cntp, r1, W1l, b1, W2r)


def _final_call(aggp, cntp, r2, W2l, b2):
    def body(a_ref, c_ref, r_ref, wl_ref, b_ref, o_ref):
        av = a_ref[...]
        cv = c_ref[...]
        a = jnp.concatenate([av[0], av[1]], axis=1)
        cnt = jnp.sum(cv, axis=1, keepdims=True)
        mean = a / jnp.maximum(cnt, 1.0)
        v = lax.dot_general(mean, wl_ref[...], (((1,), (1,)), ((), ())),
                            preferred_element_type=jnp.float32)
        o_ref[...] = _elu(v + b_ref[...] + r_ref[...])

    return pl.pallas_call(
        body,
        grid=(NPAD // BLK,),
        in_specs=[pl.BlockSpec((NC, BLK, FW), lambda i: (0, i, 0)),
                  pl.BlockSpec((BLK, NW), lambda i: (i, 0)),
                  pl.BlockSpec((BLK, H), lambda i: (i, 0)),
                  pl.BlockSpec((H, H), lambda i: (0, 0)),
                  pl.BlockSpec((1, H), lambda i: (0, 0))],
        out_specs=pl.BlockSpec((BLK, H), lambda i: (i, 0)),
        out_shape=jax.ShapeDtypeStruct((NPAD, H), jnp.float32),
        compiler_params=pltpu.CompilerParams(
            dimension_semantics=("parallel",)),
    )(aggp, cntp, r2, W2l, b2)


def kernel(x, knn_edge_index, W1l, b1, W1r, W2l, b2, W2r):
    src = knn_edge_index[0].astype(jnp.int32)
    dst = knn_edge_index[1].astype(jnp.int32)
    pad = EPAD - E
    srcp = jnp.concatenate([src, jnp.zeros((pad,), jnp.int32)]).reshape(
        NS * CPT, CW)
    dstp = jnp.concatenate([dst, jnp.full((pad,), N, jnp.int32)]).reshape(
        NS * CPT, CW)
    xp = jnp.pad(x, ((0, 0), (0, NPAD - N)))
    eye = jnp.eye(D, dtype=jnp.float32)
    z64 = jnp.zeros((CW, FW), jnp.float32)

    tlo, thi, r1 = _prep_call(xp, eye, W1r)
    agg1, cnt1 = _sc_agg_counts(tlo, thi, srcp, dstp, z64)
    cnt1t = cnt1.reshape(NW, NPAD).T
    hlo, hhi, r2 = _mid_call(agg1, cnt1t, r1, W1l, b1.reshape(1, H), W2r)
    agg2 = _sc_agg_plain(hlo, hhi, srcp, dstp, z64)
    o = _final_call(agg2, cnt1t, r2, W2l, b2.reshape(1, H))
    return o[:N]
